# baseline (speedup 1.0000x reference)
import jax
import jax.numpy as jnp
from jax import lax
from jax.experimental import pallas as pl
from jax.experimental.pallas import tpu as pltpu
from jax.experimental.pallas import tpu_sc as plsc

CB = 1024
NQ = 8
NC, NS, L = 2, 16, 16
NW = NC * NS
H = NQ * CB
UNROLL = 16

CE_W = 1.0
COMMIT_W = 0.25
DIV_W = 0.1


def _sc_hist_body(idx_hbm, out_hbm, chunk_v, hist_v, acc_v, sem0, sem1):
    chunk = chunk_v.shape[0]
    half = chunk // 2
    wid = lax.axis_index("s") * NC + lax.axis_index("c")
    base = wid * chunk
    dma0 = pltpu.async_copy(
        idx_hbm.at[pl.ds(base, half)], chunk_v.at[pl.ds(0, half)], sem0)
    dma1 = pltpu.async_copy(
        idx_hbm.at[pl.ds(base + half, half)], chunk_v.at[pl.ds(half, half)],
        sem1)

    zeros = jnp.zeros((L,), jnp.float32)

    def zbody(i, c):
        for u in range(UNROLL):
            hist_v[pl.ds((i * UNROLL + u) * L, L)] = zeros
        return c

    lax.fori_loop(0, (2 * H) // (L * UNROLL), zbody, 0)

    lane = lax.iota(jnp.int32, L)
    region = (lane & (NQ - 1)) * CB + (lane >> 3) * H
    ones = jnp.ones((L,), jnp.float32)

    def body(i, c):
        for u in range(UNROLL):
            tok = chunk_v[pl.ds((i * UNROLL + u) * L, L)]
            plsc.addupdate_scatter(hist_v, [tok + region], ones)
        return c

    n_half = half // (L * UNROLL)
    dma0.wait()
    lax.fori_loop(0, n_half, body, 0)
    dma1.wait()
    lax.fori_loop(n_half, 2 * n_half, body, 0)

    def cbody(i, c):
        for u in range(UNROLL):
            j = (i * UNROLL + u) * L
            acc_v[pl.ds(j, L)] = hist_v[pl.ds(j, L)] + hist_v[pl.ds(H + j, L)]
        return c

    lax.fori_loop(0, H // (L * UNROLL), cbody, 0)
    pltpu.sync_copy(acc_v, out_hbm.at[pl.ds(wid * H, H)])


def _entropy_body(lm_ref, co_ref, parts_ref, ce_ref, com_ref, div_ref, tot_ref):
    acc = parts_ref[0:64, :]
    for p in range(1, NW):
        acc = acc + parts_ref[p * 64:(p + 1) * 64, :]
    s = jnp.float32(0.0)
    for q in range(NQ):
        blk = acc[q * 8:(q + 1) * 8, :]
        prob = blk / jnp.sum(blk)
        s = s + jnp.sum(prob * jnp.log(prob + 1e-8))
    ce = lm_ref[0, 0] * CE_W
    co = co_ref[0, 0] * COMMIT_W
    div = (s / NQ) * DIV_W
    ce_ref[0, 0] = ce
    com_ref[0, 0] = co
    div_ref[0, 0] = div
    tot_ref[0, 0] = ce + co + div


def kernel(lm_loss, rvq_commitment_loss, rvq_indices):
    b, sl, nq = rvq_indices.shape
    n_total = b * sl * nq
    chunk = n_total // NW
    flat = rvq_indices.reshape(n_total)

    hist_fn = pl.kernel(
        _sc_hist_body,
        mesh=plsc.VectorSubcoreMesh(core_axis_name="c", subcore_axis_name="s"),
        out_type=jax.ShapeDtypeStruct((NW * H,), jnp.float32),
        scratch_types=[
            pltpu.VMEM((chunk,), jnp.int32),
            pltpu.VMEM((2 * H,), jnp.float32),
            pltpu.VMEM((H,), jnp.float32),
            pltpu.SemaphoreType.DMA,
            pltpu.SemaphoreType.DMA,
        ],
        compiler_params=pltpu.CompilerParams(needs_layout_passes=False),
    )
    partials = hist_fn(flat)

    parts2 = partials.reshape(NW * 64, 128)
    lm = jnp.asarray(lm_loss, jnp.float32).reshape(1, 1)
    co = jnp.asarray(rvq_commitment_loss, jnp.float32).reshape(1, 1)

    scalar = jax.ShapeDtypeStruct((1, 1), jnp.float32)
    ce, com, div, tot = pl.pallas_call(
        _entropy_body,
        out_shape=[scalar, scalar, scalar, scalar],
        in_specs=[
            pl.BlockSpec(memory_space=pltpu.SMEM),
            pl.BlockSpec(memory_space=pltpu.SMEM),
            pl.BlockSpec(memory_space=pltpu.VMEM),
        ],
        out_specs=[pl.BlockSpec(memory_space=pltpu.SMEM)] * 4,
    )(lm, co, parts2)

    return (
        ce.reshape(()),
        com.reshape(()),
        div.reshape(()),
        tot.reshape(()),
    )


# speedup vs baseline: 3.9498x; 3.9498x over previous
import jax
import jax.numpy as jnp
from jax import lax
from jax.experimental import pallas as pl
from jax.experimental.pallas import tpu as pltpu
from jax.experimental.pallas import tpu_sc as plsc

CB = 1024
NQ = 8
NC, NS, L = 2, 16, 16
NW = NC * NS
H = NQ * CB
UNROLL = 16

CE_W = 1.0
COMMIT_W = 0.25
DIV_W = 0.1


def _sc_hist_body(idx_hbm, out_hbm, chunk_v, hist_v, sem0, sem1):
    chunk = chunk_v.shape[0]
    half = chunk // 2
    wid = lax.axis_index("s") * NC + lax.axis_index("c")
    base = wid * chunk
    dma0 = pltpu.async_copy(
        idx_hbm.at[pl.ds(base, half)], chunk_v.at[pl.ds(0, half)], sem0)
    dma1 = pltpu.async_copy(
        idx_hbm.at[pl.ds(base + half, half)], chunk_v.at[pl.ds(half, half)],
        sem1)

    zeros = jnp.zeros((L,), jnp.float32)

    def zbody(i, c):
        for u in range(UNROLL):
            hist_v[pl.ds((i * UNROLL + u) * L, L)] = zeros
        return c

    lax.fori_loop(0, H // (L * UNROLL), zbody, 0)

    ones = jnp.ones((L,), jnp.float32)

    def body(i, c):
        toks = [chunk_v[pl.ds((i * UNROLL + u) * L, L)] for u in range(UNROLL)]
        bins = []
        for u in range(UNROLL):
            q = ((i * UNROLL + u) >> 3) & (NQ - 1)
            bins.append(toks[u] + jnp.broadcast_to(q * CB, (L,)).astype(jnp.int32))
        for u in range(UNROLL):
            plsc.addupdate_scatter(hist_v, [bins[u]], ones)
        return c

    n_half = half // (L * UNROLL)
    dma0.wait()
    lax.fori_loop(0, n_half, body, 0)
    dma1.wait()
    lax.fori_loop(n_half, 2 * n_half, body, 0)

    pltpu.sync_copy(hist_v, out_hbm.at[pl.ds(wid * H, H)])


def _entropy_body(lm_ref, co_ref, parts_ref, ce_ref, com_ref, div_ref, tot_ref):
    acc = parts_ref[0:64, :]
    for p in range(1, NW):
        acc = acc + parts_ref[p * 64:(p + 1) * 64, :]
    s = jnp.float32(0.0)
    for q in range(NQ):
        blk = acc[q * 8:(q + 1) * 8, :]
        prob = blk / jnp.sum(blk)
        s = s + jnp.sum(prob * jnp.log(prob + 1e-8))
    ce = lm_ref[0, 0] * CE_W
    co = co_ref[0, 0] * COMMIT_W
    div = (s / NQ) * DIV_W
    ce_ref[0, 0] = ce
    com_ref[0, 0] = co
    div_ref[0, 0] = div
    tot_ref[0, 0] = ce + co + div


def kernel(lm_loss, rvq_commitment_loss, rvq_indices):
    b, sl, nq = rvq_indices.shape
    n_total = b * sl * nq
    chunk = n_total // NW
    flat = (
        rvq_indices.reshape(b, sl // 128, 128, nq)
        .transpose(0, 1, 3, 2)
        .reshape(n_total)
    )

    hist_fn = pl.kernel(
        _sc_hist_body,
        mesh=plsc.VectorSubcoreMesh(core_axis_name="c", subcore_axis_name="s"),
        out_type=jax.ShapeDtypeStruct((NW * H,), jnp.float32),
        scratch_types=[
            pltpu.VMEM((chunk,), jnp.int32),
            pltpu.VMEM((H,), jnp.float32),
            pltpu.SemaphoreType.DMA,
            pltpu.SemaphoreType.DMA,
        ],
        compiler_params=pltpu.CompilerParams(needs_layout_passes=False),
    )
    partials = hist_fn(flat)

    parts2 = partials.reshape(NW * 64, 128)
    lm = jnp.asarray(lm_loss, jnp.float32).reshape(1, 1)
    co = jnp.asarray(rvq_commitment_loss, jnp.float32).reshape(1, 1)

    scalar = jax.ShapeDtypeStruct((1, 1), jnp.float32)
    ce, com, div, tot = pl.pallas_call(
        _entropy_body,
        out_shape=[scalar, scalar, scalar, scalar],
        in_specs=[
            pl.BlockSpec(memory_space=pltpu.SMEM),
            pl.BlockSpec(memory_space=pltpu.SMEM),
            pl.BlockSpec(memory_space=pltpu.VMEM),
        ],
        out_specs=[pl.BlockSpec(memory_space=pltpu.SMEM)] * 4,
    )(lm, co, parts2)

    return (
        ce.reshape(()),
        com.reshape(()),
        div.reshape(()),
        tot.reshape(()),
    )
